# baseline (speedup 1.0000x reference)
import functools

import jax
import jax.numpy as jnp
from jax import lax
from jax.experimental import pallas as pl
from jax.experimental.pallas import tpu as pltpu
from jax.experimental.pallas import tpu_sc as plsc

N = 10000
E = 320000
D = 128
G = 64
DH = D // 2

NC = 2
NS = 16
NW = NC * NS

CHUNK = 128
EPS_REAL = E // NS
NJE = 160
PADS = NJE * CHUNK - EPS_REAL
NJ = NJE // 2

N_PAD = 10240
TRASH = N_PAD - N
RPT = N_PAD // NS

NB = 4

R = 2048

_mesh = plsc.VectorSubcoreMesh(core_axis_name="c", subcore_axis_name="s")


@functools.partial(
    pl.kernel,
    out_type=jax.ShapeDtypeStruct((NC * N_PAD,), jnp.float32),
    mesh=_mesh,
    scratch_types=[
        pltpu.VMEM((NJ, CHUNK), jnp.int32),
        pltpu.VMEM((CHUNK,), jnp.float32),
        pltpu.VMEM((RPT,), jnp.float32),
        pltpu.VMEM_SHARED((N_PAD,), jnp.float32),
    ],
)
def _deg_kernel(dst_hbm, out_hbm, idx_v, ones_v, zb_v, acc_sh):
    c = lax.axis_index("c")
    s = lax.axis_index("s")
    w = c * NS + s

    def zf(i, _):
        zb_v[pl.ds(i * 16, 16)] = jnp.zeros((16,), jnp.float32)
        return 0
    lax.fori_loop(0, RPT // 16, zf, 0)

    def of(i, _):
        ones_v[pl.ds(i * 16, 16)] = jnp.ones((16,), jnp.float32)
        return 0
    lax.fori_loop(0, CHUNK // 16, of, 0)

    pltpu.sync_copy(zb_v, acc_sh.at[pl.ds(s * RPT, RPT)])
    plsc.subcore_barrier()

    pltpu.sync_copy(dst_hbm.at[pl.ds(w * NJ, NJ)], idx_v)

    def body(j, _):
        pltpu.sync_copy(ones_v, acc_sh.at[idx_v.at[j]], add=True)
        return 0
    lax.fori_loop(0, NJ, body, 0)

    plsc.subcore_barrier()
    pltpu.sync_copy(acc_sh.at[pl.ds(s * RPT, RPT)],
                    out_hbm.at[pl.ds(c * N_PAD + s * RPT, RPT)])


PASSES = 10
CPP = NJE // PASSES
LG = NB // 2


@functools.partial(
    pl.kernel,
    out_type=jax.ShapeDtypeStruct((NC * N_PAD, DH), jnp.float32),
    mesh=_mesh,
    scratch_types=[
        [pltpu.VMEM((CPP, CHUNK), jnp.int32)] * 2,
        [pltpu.VMEM((CPP, CHUNK), jnp.int32)] * 2,
        [pltpu.VMEM((CHUNK, DH), jnp.float32)] * NB,
        [pltpu.SemaphoreType.DMA] * 2,
        [pltpu.SemaphoreType.DMA] * 2,
        pltpu.SemaphoreType.DMA,
        [pltpu.SemaphoreType.DMA] * NB,
        [pltpu.SemaphoreType.DMA] * NB,
        pltpu.VMEM_SHARED((N_PAD, DH), jnp.float32),
        pltpu.VMEM_SHARED((N_PAD, DH), jnp.float32),
    ],
    compiler_params=pltpu.CompilerParams(use_tc_tiling_on_sc=False),
)
def _agg_kernel(hs_hbm, src_hbm, dst_hbm, out_hbm, srcs, dsts, rows,
                issem, idsem, hsem, gsem, ssem, hs_sh, acc_sh):
    c = lax.axis_index("c")
    s = lax.axis_index("s")

    hscpy = pltpu.make_async_copy(
        hs_hbm.at[pl.ds(c * N_PAD + s * RPT, RPT)],
        hs_sh.at[pl.ds(s * RPT, RPT)], hsem)
    hscpy.start()

    def igs(p, e):
        row0 = s * NJE + p * CPP
        pltpu.async_copy(src_hbm.at[pl.ds(row0, CPP)], srcs[e], issem[e])
        pltpu.async_copy(dst_hbm.at[pl.ds(row0, CPP)], dsts[e], idsem[e])

    def igw(p, e):
        row0 = s * NJE + p * CPP
        pltpu.make_async_copy(src_hbm.at[pl.ds(row0, CPP)], srcs[e],
                              issem[e]).wait()
        pltpu.make_async_copy(dst_hbm.at[pl.ds(row0, CPP)], dsts[e],
                              idsem[e]).wait()

    igs(0, 0)

    def zf(i, _):
        r = i // (DH // 16)
        col = (i % (DH // 16)) * 16
        rows[0][r, pl.ds(col, 16)] = jnp.zeros((16,), jnp.float32)
        return 0
    lax.fori_loop(0, CHUNK * DH // 16, zf, 0)

    def zcopy(t, _):
        pltpu.sync_copy(rows[0], acc_sh.at[pl.ds(s * RPT + t * CHUNK, CHUNK)])
        return 0
    lax.fori_loop(0, RPT // CHUNK, zcopy, 0)

    hscpy.wait()
    plsc.subcore_barrier()

    for p in range(PASSES):
        e = p % 2
        igw(p, e)
        if p + 1 < PASSES:
            igs(p + 1, (p + 1) % 2)

        def gs(k, r, e=e):
            pltpu.async_copy(hs_sh.at[srcs[e].at[k]], rows[r], gsem[r])

        def gw(k, r, e=e):
            pltpu.make_async_copy(hs_sh.at[srcs[e].at[k]], rows[r],
                                  gsem[r]).wait()

        def ss(k, r, e=e):
            pltpu.async_copy(rows[r], acc_sh.at[dsts[e].at[k]], ssem[r],
                             add=True)

        def sw(k, r, e=e):
            pltpu.make_async_copy(rows[r], acc_sh.at[dsts[e].at[k]],
                                  ssem[r]).wait()

        gs(0, 0)
        gs(1, 1)

        def ring(i, _):
            for b in range(NB):
                k = NB * i + b
                rb = (b + LG) % NB

                @pl.when(k >= NB - LG)
                def _():
                    sw(k - (NB - LG), rb)

                @pl.when(k + LG < CPP)
                def _():
                    gs(k + LG, rb)

                gw(k, b)
                ss(k, b)
            return 0
        lax.fori_loop(0, CPP // NB, ring, 0)

        for k in range(CPP - (NB - LG), CPP):
            sw(k, k % NB)

    plsc.subcore_barrier()

    def wb(t, _):
        pltpu.sync_copy(acc_sh.at[pl.ds(s * RPT + t * CHUNK, CHUNK)],
                        out_hbm.at[pl.ds(c * N_PAD + s * RPT + t * CHUNK, CHUNK)])
        return 0
    lax.fori_loop(0, RPT // CHUNK, wb, 0)


def _tc1_body(x_ref, w_ref, degp_ref, o_ref):
    j = pl.program_id(0)
    deg = degp_ref[0, pl.ds(j * R, R)] + degp_ref[1, pl.ds(j * R, R)] + 1.0
    dis = lax.rsqrt(deg)[:, None]
    h = jnp.dot(x_ref[...], w_ref[...], preferred_element_type=jnp.float32)
    h = h * dis
    o_ref[0] = h[:, :DH]
    o_ref[1] = h[:, DH:]


def _tc_mid_body(aggp_ref, hs_ref, degp_ref, w_ref, b_ref, o_ref):
    j = pl.program_id(0)
    deg = degp_ref[0, pl.ds(j * R, R)] + degp_ref[1, pl.ds(j * R, R)] + 1.0
    dis = lax.rsqrt(deg)[:, None]
    agg = jnp.concatenate([aggp_ref[0], aggp_ref[1]], axis=1)
    hs = jnp.concatenate([hs_ref[0], hs_ref[1]], axis=1)
    z = (agg + hs) * dis + b_ref[...]
    a = jnp.maximum(z, 0.0)
    h = jnp.dot(a, w_ref[...], preferred_element_type=jnp.float32) * dis
    o_ref[0] = h[:, :DH]
    o_ref[1] = h[:, DH:]


def _tc_final_body(aggp_ref, hs_ref, degp_ref, b_ref, batch_ref, o_ref):
    j = pl.program_id(0)
    deg = degp_ref[0, pl.ds(j * R, R)] + degp_ref[1, pl.ds(j * R, R)] + 1.0
    dis = lax.rsqrt(deg)[:, None]
    agg = jnp.concatenate([aggp_ref[0], aggp_ref[1]], axis=1)
    hs = jnp.concatenate([hs_ref[0], hs_ref[1]], axis=1)
    z = (agg + hs) * dis + b_ref[...]
    bb = batch_ref[...]
    oh = (bb == lax.broadcasted_iota(jnp.int32, (1, G), 1)).astype(jnp.float32)
    contrib = lax.dot_general(oh, z, (((0,), (0,)), ((), ())),
                              preferred_element_type=jnp.float32)

    @pl.when(j == 0)
    def _():
        o_ref[...] = jnp.zeros_like(o_ref)

    o_ref[...] += contrib


_x_spec = pl.BlockSpec((R, D), lambda j: (j, 0))
_w_spec = pl.BlockSpec((D, D), lambda j: (0, 0))
_degp_spec = pl.BlockSpec((2, N_PAD), lambda j: (0, 0))
_split_spec = pl.BlockSpec((2, R, DH), lambda j: (0, j, 0))
_b_spec = pl.BlockSpec((1, D), lambda j: (0, 0))

_split_shape = jax.ShapeDtypeStruct((2, N_PAD, DH), jnp.float32)

_tc1 = pl.pallas_call(
    _tc1_body,
    grid=(N_PAD // R,),
    in_specs=[_x_spec, _w_spec, _degp_spec],
    out_specs=_split_spec,
    out_shape=_split_shape,
)

_tc_mid = pl.pallas_call(
    _tc_mid_body,
    grid=(N_PAD // R,),
    in_specs=[_split_spec, _split_spec, _degp_spec, _w_spec, _b_spec],
    out_specs=_split_spec,
    out_shape=_split_shape,
)

_tc_final = pl.pallas_call(
    _tc_final_body,
    grid=(N_PAD // R,),
    in_specs=[_split_spec, _split_spec, _degp_spec, _b_spec,
              pl.BlockSpec((R, 1), lambda j: (j, 0))],
    out_specs=pl.BlockSpec((G, D), lambda j: (0, 0)),
    out_shape=jax.ShapeDtypeStruct((G, D), jnp.float32),
)


def kernel(x, edge_index, batch, W1, b1, W2, b2, W3, b3):
    src = edge_index[0].reshape(NS, EPS_REAL)
    dst = edge_index[1].reshape(NS, EPS_REAL)
    iw = jnp.arange(NS, dtype=jnp.int32)[:, None]
    ip = jnp.arange(PADS, dtype=jnp.int32)[None, :]
    pad_src = (iw * 613 + ip * 37) % N
    pad_dst = N + (iw * 7 + ip) % TRASH
    src_p = jnp.concatenate([src, pad_src], axis=1).reshape(NS * NJE, CHUNK)
    dst_p = jnp.concatenate([dst, pad_dst], axis=1).reshape(NS * NJE, CHUNK)

    degp = _deg_kernel(dst_p).reshape(NC, N_PAD)
    x_p = jnp.pad(x, ((0, N_PAD - N), (0, 0)))
    batch_p = jnp.pad(batch, (0, N_PAD - N), constant_values=G)
    hs1 = _tc1(x_p, W1, degp)
    agg1 = _agg_kernel(hs1.reshape(NC * N_PAD, DH), src_p, dst_p)
    hs2 = _tc_mid(agg1.reshape(2, N_PAD, DH), hs1, degp, W2, b1.reshape(1, D))
    agg2 = _agg_kernel(hs2.reshape(NC * N_PAD, DH), src_p, dst_p)
    hs3 = _tc_mid(agg2.reshape(2, N_PAD, DH), hs2, degp, W3, b2.reshape(1, D))
    agg3 = _agg_kernel(hs3.reshape(NC * N_PAD, DH), src_p, dst_p)
    out = _tc_final(agg3.reshape(2, N_PAD, DH), hs3, degp,
                    b3.reshape(1, D), batch_p.reshape(N_PAD, 1))
    return out


# speedup vs baseline: 1.2369x; 1.2369x over previous
import functools

import jax
import jax.numpy as jnp
from jax import lax
from jax.experimental import pallas as pl
from jax.experimental.pallas import tpu as pltpu
from jax.experimental.pallas import tpu_sc as plsc

N = 10000
E = 320000
D = 128
G = 64
DH = D // 2

NC = 2
NS = 16
NW = NC * NS

CHUNK = 128
EPS_REAL = E // NS
NJE = 160
PADS = NJE * CHUNK - EPS_REAL
NJ = NJE // 2

N_PAD = 10240
TRASH = N_PAD - N
RPT = N_PAD // NS

NB = 4

R = 2048

_mesh = plsc.VectorSubcoreMesh(core_axis_name="c", subcore_axis_name="s")


@functools.partial(
    pl.kernel,
    out_type=jax.ShapeDtypeStruct((NC * N_PAD,), jnp.float32),
    mesh=_mesh,
    scratch_types=[
        pltpu.VMEM((NJ, CHUNK), jnp.int32),
        pltpu.VMEM((CHUNK,), jnp.float32),
        pltpu.VMEM((RPT,), jnp.float32),
        pltpu.VMEM_SHARED((N_PAD,), jnp.float32),
    ],
)
def _deg_kernel(dst_hbm, out_hbm, idx_v, ones_v, zb_v, acc_sh):
    c = lax.axis_index("c")
    s = lax.axis_index("s")
    w = c * NS + s

    def zf(i, _):
        zb_v[pl.ds(i * 16, 16)] = jnp.zeros((16,), jnp.float32)
        return 0
    lax.fori_loop(0, RPT // 16, zf, 0)

    def of(i, _):
        ones_v[pl.ds(i * 16, 16)] = jnp.ones((16,), jnp.float32)
        return 0
    lax.fori_loop(0, CHUNK // 16, of, 0)

    pltpu.sync_copy(zb_v, acc_sh.at[pl.ds(s * RPT, RPT)])
    plsc.subcore_barrier()

    pltpu.sync_copy(dst_hbm.at[pl.ds(w * NJ, NJ)], idx_v)

    def body(j, _):
        pltpu.sync_copy(ones_v, acc_sh.at[idx_v.at[j]], add=True)
        return 0
    lax.fori_loop(0, NJ, body, 0)

    plsc.subcore_barrier()
    pltpu.sync_copy(acc_sh.at[pl.ds(s * RPT, RPT)],
                    out_hbm.at[pl.ds(c * N_PAD + s * RPT, RPT)])


@functools.partial(
    pl.kernel,
    out_type=jax.ShapeDtypeStruct((NC * N_PAD, DH), jnp.float32),
    mesh=_mesh,
    scratch_types=[
        pltpu.VMEM((NJE, CHUNK), jnp.int32),
        pltpu.VMEM((NJE, CHUNK), jnp.int32),
        [pltpu.VMEM((CHUNK, DH), jnp.float32)] * NB,
        pltpu.VMEM((CHUNK, DH), jnp.float32),
        [pltpu.SemaphoreType.DMA] * 2,
        [pltpu.SemaphoreType.DMA] * NB,
        [pltpu.SemaphoreType.DMA] * NB,
        pltpu.VMEM_SHARED((N_PAD, DH), jnp.float32),
    ],
    compiler_params=pltpu.CompilerParams(use_tc_tiling_on_sc=False),
)
def _agg_kernel(hs_hbm, src_hbm, dst_hbm, out_hbm, src_v, dst_v, rows, zb_v,
                isem, gsem, ssem, acc_sh):
    c = lax.axis_index("c")
    s = lax.axis_index("s")

    scpy = pltpu.make_async_copy(
        src_hbm.at[pl.ds((c * NS + s) * NJE, NJE)], src_v, isem[0])
    scpy.start()
    dcpy = pltpu.make_async_copy(
        dst_hbm.at[pl.ds(s * NJE, NJE)], dst_v, isem[1])
    dcpy.start()

    def gs(k, r):
        pltpu.async_copy(hs_hbm.at[src_v.at[k]], rows[r], gsem[r])

    def gw(k, r):
        pltpu.make_async_copy(hs_hbm.at[src_v.at[k]], rows[r], gsem[r]).wait()

    def ss(k, r):
        pltpu.async_copy(rows[r], acc_sh.at[dst_v.at[k]], ssem[r], add=True)

    def sw(k, r):
        pltpu.make_async_copy(rows[r], acc_sh.at[dst_v.at[k]], ssem[r]).wait()

    def zf(i, _):
        r = i // (DH // 16)
        col = (i % (DH // 16)) * 16
        zb_v[r, pl.ds(col, 16)] = jnp.zeros((16,), jnp.float32)
        return 0
    lax.fori_loop(0, CHUNK * DH // 16, zf, 0)

    scpy.wait()
    dcpy.wait()
    gs(0, 0)
    gs(1, 1)

    def zcopy(t, _):
        pltpu.sync_copy(zb_v, acc_sh.at[pl.ds(s * RPT + t * CHUNK, CHUNK)])
        return 0
    lax.fori_loop(0, RPT // CHUNK, zcopy, 0)
    plsc.subcore_barrier()

    LG = NB // 2

    def ring(i, _):
        for b in range(NB):
            k = NB * i + b
            rb = (b + LG) % NB

            @pl.when(k >= NB - LG)
            def _():
                sw(k - (NB - LG), rb)

            @pl.when(k + LG < NJE)
            def _():
                gs(k + LG, rb)

            gw(k, b)
            ss(k, b)
        return 0
    lax.fori_loop(0, NJE // NB, ring, 0)

    for k in range(NJE - (NB - LG), NJE):
        sw(k, k % NB)

    plsc.subcore_barrier()

    def wb(t, _):
        pltpu.sync_copy(acc_sh.at[pl.ds(s * RPT + t * CHUNK, CHUNK)],
                        out_hbm.at[pl.ds(c * N_PAD + s * RPT + t * CHUNK, CHUNK)])
        return 0
    lax.fori_loop(0, RPT // CHUNK, wb, 0)


def _tc1_body(x_ref, w_ref, degp_ref, o_ref):
    j = pl.program_id(0)
    deg = degp_ref[0, pl.ds(j * R, R)] + degp_ref[1, pl.ds(j * R, R)] + 1.0
    dis = lax.rsqrt(deg)[:, None]
    h = jnp.dot(x_ref[...], w_ref[...], preferred_element_type=jnp.float32)
    h = h * dis
    o_ref[0] = h[:, :DH]
    o_ref[1] = h[:, DH:]


def _tc_mid_body(aggp_ref, hs_ref, degp_ref, w_ref, b_ref, o_ref):
    j = pl.program_id(0)
    deg = degp_ref[0, pl.ds(j * R, R)] + degp_ref[1, pl.ds(j * R, R)] + 1.0
    dis = lax.rsqrt(deg)[:, None]
    agg = jnp.concatenate([aggp_ref[0], aggp_ref[1]], axis=1)
    hs = jnp.concatenate([hs_ref[0], hs_ref[1]], axis=1)
    z = (agg + hs) * dis + b_ref[...]
    a = jnp.maximum(z, 0.0)
    h = jnp.dot(a, w_ref[...], preferred_element_type=jnp.float32) * dis
    o_ref[0] = h[:, :DH]
    o_ref[1] = h[:, DH:]


def _tc_final_body(aggp_ref, hs_ref, degp_ref, b_ref, batch_ref, o_ref):
    j = pl.program_id(0)
    deg = degp_ref[0, pl.ds(j * R, R)] + degp_ref[1, pl.ds(j * R, R)] + 1.0
    dis = lax.rsqrt(deg)[:, None]
    agg = jnp.concatenate([aggp_ref[0], aggp_ref[1]], axis=1)
    hs = jnp.concatenate([hs_ref[0], hs_ref[1]], axis=1)
    z = (agg + hs) * dis + b_ref[...]
    bb = batch_ref[...]
    oh = (bb == lax.broadcasted_iota(jnp.int32, (1, G), 1)).astype(jnp.float32)
    contrib = lax.dot_general(oh, z, (((0,), (0,)), ((), ())),
                              preferred_element_type=jnp.float32)

    @pl.when(j == 0)
    def _():
        o_ref[...] = jnp.zeros_like(o_ref)

    o_ref[...] += contrib


_x_spec = pl.BlockSpec((R, D), lambda j: (j, 0))
_w_spec = pl.BlockSpec((D, D), lambda j: (0, 0))
_degp_spec = pl.BlockSpec((2, N_PAD), lambda j: (0, 0))
_split_spec = pl.BlockSpec((2, R, DH), lambda j: (0, j, 0))
_b_spec = pl.BlockSpec((1, D), lambda j: (0, 0))

_split_shape = jax.ShapeDtypeStruct((2, N_PAD, DH), jnp.float32)

_tc1 = pl.pallas_call(
    _tc1_body,
    grid=(N_PAD // R,),
    in_specs=[_x_spec, _w_spec, _degp_spec],
    out_specs=_split_spec,
    out_shape=_split_shape,
)

_tc_mid = pl.pallas_call(
    _tc_mid_body,
    grid=(N_PAD // R,),
    in_specs=[_split_spec, _split_spec, _degp_spec, _w_spec, _b_spec],
    out_specs=_split_spec,
    out_shape=_split_shape,
)

_tc_final = pl.pallas_call(
    _tc_final_body,
    grid=(N_PAD // R,),
    in_specs=[_split_spec, _split_spec, _degp_spec, _b_spec,
              pl.BlockSpec((R, 1), lambda j: (j, 0))],
    out_specs=pl.BlockSpec((G, D), lambda j: (0, 0)),
    out_shape=jax.ShapeDtypeStruct((G, D), jnp.float32),
)


def kernel(x, edge_index, batch, W1, b1, W2, b2, W3, b3):
    src = edge_index[0].reshape(NS, EPS_REAL)
    dst = edge_index[1].reshape(NS, EPS_REAL)
    iw = jnp.arange(NS, dtype=jnp.int32)[:, None]
    ip = jnp.arange(PADS, dtype=jnp.int32)[None, :]
    pad_src = (iw * 613 + ip * 37) % N
    pad_dst = N + (iw * 7 + ip) % TRASH
    src_p = jnp.concatenate([src, pad_src], axis=1).reshape(NS * NJE, CHUNK)
    dst_p = jnp.concatenate([dst, pad_dst], axis=1).reshape(NS * NJE, CHUNK)
    src_p2 = jnp.concatenate([src_p, src_p + N_PAD], axis=0)

    degp = _deg_kernel(dst_p).reshape(NC, N_PAD)
    x_p = jnp.pad(x, ((0, N_PAD - N), (0, 0)))
    batch_p = jnp.pad(batch, (0, N_PAD - N), constant_values=G)
    hs1 = _tc1(x_p, W1, degp)
    agg1 = _agg_kernel(hs1.reshape(NC * N_PAD, DH), src_p2, dst_p)
    hs2 = _tc_mid(agg1.reshape(2, N_PAD, DH), hs1, degp, W2, b1.reshape(1, D))
    agg2 = _agg_kernel(hs2.reshape(NC * N_PAD, DH), src_p2, dst_p)
    hs3 = _tc_mid(agg2.reshape(2, N_PAD, DH), hs2, degp, W3, b2.reshape(1, D))
    agg3 = _agg_kernel(hs3.reshape(NC * N_PAD, DH), src_p2, dst_p)
    out = _tc_final(agg3.reshape(2, N_PAD, DH), hs3, degp,
                    b3.reshape(1, D), batch_p.reshape(N_PAD, 1))
    return out


# speedup vs baseline: 1.2990x; 1.0502x over previous
import functools

import jax
import jax.numpy as jnp
from jax import lax
from jax.experimental import pallas as pl
from jax.experimental.pallas import tpu as pltpu
from jax.experimental.pallas import tpu_sc as plsc

N = 10000
E = 320000
D = 128
G = 64
DH = D // 2

NC = 2
NS = 16
NW = NC * NS

CHUNK = 128
EPS_REAL = E // NS
NJE = 160
PADS = NJE * CHUNK - EPS_REAL
NJ = NJE // 2

N_PAD = 10240
TRASH = N_PAD - N
RPT = N_PAD // NS

NB = 5
ZR = 64

R = 2048

_mesh = plsc.VectorSubcoreMesh(core_axis_name="c", subcore_axis_name="s")


@functools.partial(
    pl.kernel,
    out_type=jax.ShapeDtypeStruct((NC * N_PAD,), jnp.float32),
    mesh=_mesh,
    scratch_types=[
        pltpu.VMEM((NJ, CHUNK), jnp.int32),
        pltpu.VMEM((CHUNK,), jnp.float32),
        pltpu.VMEM((RPT,), jnp.float32),
        pltpu.VMEM_SHARED((N_PAD,), jnp.float32),
    ],
)
def _deg_kernel(dst_hbm, out_hbm, idx_v, ones_v, zb_v, acc_sh):
    c = lax.axis_index("c")
    s = lax.axis_index("s")
    w = c * NS + s

    def zf(i, _):
        zb_v[pl.ds(i * 16, 16)] = jnp.zeros((16,), jnp.float32)
        return 0
    lax.fori_loop(0, RPT // 16, zf, 0)

    def of(i, _):
        ones_v[pl.ds(i * 16, 16)] = jnp.ones((16,), jnp.float32)
        return 0
    lax.fori_loop(0, CHUNK // 16, of, 0)

    pltpu.sync_copy(zb_v, acc_sh.at[pl.ds(s * RPT, RPT)])
    plsc.subcore_barrier()

    pltpu.sync_copy(dst_hbm.at[pl.ds(w * NJ, NJ)], idx_v)

    def body(j, _):
        pltpu.sync_copy(ones_v, acc_sh.at[idx_v.at[j]], add=True)
        return 0
    lax.fori_loop(0, NJ, body, 0)

    plsc.subcore_barrier()
    pltpu.sync_copy(acc_sh.at[pl.ds(s * RPT, RPT)],
                    out_hbm.at[pl.ds(c * N_PAD + s * RPT, RPT)])


@functools.partial(
    pl.kernel,
    out_type=jax.ShapeDtypeStruct((NC * N_PAD, DH), jnp.float32),
    mesh=_mesh,
    scratch_types=[
        pltpu.VMEM((NJE, CHUNK), jnp.int32),
        pltpu.VMEM((NJE, CHUNK), jnp.int32),
        [pltpu.VMEM((CHUNK, DH), jnp.float32)] * NB,
        pltpu.VMEM((ZR, DH), jnp.float32),
        [pltpu.SemaphoreType.DMA] * 2,
        [pltpu.SemaphoreType.DMA] * NB,
        [pltpu.SemaphoreType.DMA] * NB,
        pltpu.VMEM_SHARED((N_PAD, DH), jnp.float32),
    ],
    compiler_params=pltpu.CompilerParams(use_tc_tiling_on_sc=False),
)
def _agg_kernel(hs_hbm, src_hbm, dst_hbm, out_hbm, src_v, dst_v, rows, zb_v,
                isem, gsem, ssem, acc_sh):
    c = lax.axis_index("c")
    s = lax.axis_index("s")

    scpy = pltpu.make_async_copy(
        src_hbm.at[pl.ds((c * NS + s) * NJE, NJE)], src_v, isem[0])
    scpy.start()
    dcpy = pltpu.make_async_copy(
        dst_hbm.at[pl.ds(s * NJE, NJE)], dst_v, isem[1])
    dcpy.start()

    def gs(k, r):
        pltpu.async_copy(hs_hbm.at[src_v.at[k]], rows[r], gsem[r])

    def gw(k, r):
        pltpu.make_async_copy(hs_hbm.at[src_v.at[k]], rows[r], gsem[r]).wait()

    def ss(k, r):
        pltpu.async_copy(rows[r], acc_sh.at[dst_v.at[k]], ssem[r], add=True)

    def sw(k, r):
        pltpu.make_async_copy(rows[r], acc_sh.at[dst_v.at[k]], ssem[r]).wait()

    def zf(i, _):
        r = i // (DH // 16)
        col = (i % (DH // 16)) * 16
        zb_v[r, pl.ds(col, 16)] = jnp.zeros((16,), jnp.float32)
        return 0
    lax.fori_loop(0, ZR * DH // 16, zf, 0)

    scpy.wait()
    dcpy.wait()
    gs(0, 0)
    gs(1, 1)

    def zcopy(t, _):
        pltpu.sync_copy(zb_v, acc_sh.at[pl.ds(s * RPT + t * ZR, ZR)])
        return 0
    lax.fori_loop(0, RPT // ZR, zcopy, 0)
    plsc.subcore_barrier()

    LG = NB // 2

    def ring(i, _):
        for b in range(NB):
            k = NB * i + b
            rb = (b + LG) % NB

            @pl.when(k >= NB - LG)
            def _():
                sw(k - (NB - LG), rb)

            @pl.when(k + LG < NJE)
            def _():
                gs(k + LG, rb)

            gw(k, b)
            ss(k, b)
        return 0
    lax.fori_loop(0, NJE // NB, ring, 0)

    for k in range(NJE - (NB - LG), NJE):
        sw(k, k % NB)

    plsc.subcore_barrier()

    def wb(t, _):
        pltpu.sync_copy(acc_sh.at[pl.ds(s * RPT + t * CHUNK, CHUNK)],
                        out_hbm.at[pl.ds(c * N_PAD + s * RPT + t * CHUNK, CHUNK)])
        return 0
    lax.fori_loop(0, RPT // CHUNK, wb, 0)


def _tc1_body(x_ref, w_ref, degp_ref, o_ref):
    j = pl.program_id(0)
    deg = degp_ref[0, pl.ds(j * R, R)] + degp_ref[1, pl.ds(j * R, R)] + 1.0
    dis = lax.rsqrt(deg)[:, None]
    h = jnp.dot(x_ref[...], w_ref[...], preferred_element_type=jnp.float32)
    h = h * dis
    o_ref[0] = h[:, :DH]
    o_ref[1] = h[:, DH:]


def _tc_mid_body(aggp_ref, hs_ref, degp_ref, w_ref, b_ref, o_ref):
    j = pl.program_id(0)
    deg = degp_ref[0, pl.ds(j * R, R)] + degp_ref[1, pl.ds(j * R, R)] + 1.0
    dis = lax.rsqrt(deg)[:, None]
    agg = jnp.concatenate([aggp_ref[0], aggp_ref[1]], axis=1)
    hs = jnp.concatenate([hs_ref[0], hs_ref[1]], axis=1)
    z = (agg + hs) * dis + b_ref[...]
    a = jnp.maximum(z, 0.0)
    h = jnp.dot(a, w_ref[...], preferred_element_type=jnp.float32) * dis
    o_ref[0] = h[:, :DH]
    o_ref[1] = h[:, DH:]


def _tc_final_body(aggp_ref, hs_ref, degp_ref, b_ref, batch_ref, o_ref):
    j = pl.program_id(0)
    deg = degp_ref[0, pl.ds(j * R, R)] + degp_ref[1, pl.ds(j * R, R)] + 1.0
    dis = lax.rsqrt(deg)[:, None]
    agg = jnp.concatenate([aggp_ref[0], aggp_ref[1]], axis=1)
    hs = jnp.concatenate([hs_ref[0], hs_ref[1]], axis=1)
    z = (agg + hs) * dis + b_ref[...]
    bb = batch_ref[...]
    oh = (bb == lax.broadcasted_iota(jnp.int32, (1, G), 1)).astype(jnp.float32)
    contrib = lax.dot_general(oh, z, (((0,), (0,)), ((), ())),
                              preferred_element_type=jnp.float32)

    @pl.when(j == 0)
    def _():
        o_ref[...] = jnp.zeros_like(o_ref)

    o_ref[...] += contrib


_x_spec = pl.BlockSpec((R, D), lambda j: (j, 0))
_w_spec = pl.BlockSpec((D, D), lambda j: (0, 0))
_degp_spec = pl.BlockSpec((2, N_PAD), lambda j: (0, 0))
_split_spec = pl.BlockSpec((2, R, DH), lambda j: (0, j, 0))
_b_spec = pl.BlockSpec((1, D), lambda j: (0, 0))

_split_shape = jax.ShapeDtypeStruct((2, N_PAD, DH), jnp.float32)

_tc1 = pl.pallas_call(
    _tc1_body,
    grid=(N_PAD // R,),
    in_specs=[_x_spec, _w_spec, _degp_spec],
    out_specs=_split_spec,
    out_shape=_split_shape,
)

_tc_mid = pl.pallas_call(
    _tc_mid_body,
    grid=(N_PAD // R,),
    in_specs=[_split_spec, _split_spec, _degp_spec, _w_spec, _b_spec],
    out_specs=_split_spec,
    out_shape=_split_shape,
)

_tc_final = pl.pallas_call(
    _tc_final_body,
    grid=(N_PAD // R,),
    in_specs=[_split_spec, _split_spec, _degp_spec, _b_spec,
              pl.BlockSpec((R, 1), lambda j: (j, 0))],
    out_specs=pl.BlockSpec((G, D), lambda j: (0, 0)),
    out_shape=jax.ShapeDtypeStruct((G, D), jnp.float32),
)


def kernel(x, edge_index, batch, W1, b1, W2, b2, W3, b3):
    src = edge_index[0].reshape(NS, EPS_REAL)
    dst = edge_index[1].reshape(NS, EPS_REAL)
    iw = jnp.arange(NS, dtype=jnp.int32)[:, None]
    ip = jnp.arange(PADS, dtype=jnp.int32)[None, :]
    pad_src = (iw * 613 + ip * 37) % N
    pad_dst = N + (iw * 7 + ip) % TRASH
    src_p = jnp.concatenate([src, pad_src], axis=1).reshape(NS * NJE, CHUNK)
    dst_p = jnp.concatenate([dst, pad_dst], axis=1).reshape(NS * NJE, CHUNK)
    src_p2 = jnp.concatenate([src_p, src_p + N_PAD], axis=0)

    degp = _deg_kernel(dst_p).reshape(NC, N_PAD)
    x_p = jnp.pad(x, ((0, N_PAD - N), (0, 0)))
    batch_p = jnp.pad(batch, (0, N_PAD - N), constant_values=G)
    hs1 = _tc1(x_p, W1, degp)
    agg1 = _agg_kernel(hs1.reshape(NC * N_PAD, DH), src_p2, dst_p)
    hs2 = _tc_mid(agg1.reshape(2, N_PAD, DH), hs1, degp, W2, b1.reshape(1, D))
    agg2 = _agg_kernel(hs2.reshape(NC * N_PAD, DH), src_p2, dst_p)
    hs3 = _tc_mid(agg2.reshape(2, N_PAD, DH), hs2, degp, W3, b2.reshape(1, D))
    agg3 = _agg_kernel(hs3.reshape(NC * N_PAD, DH), src_p2, dst_p)
    out = _tc_final(agg3.reshape(2, N_PAD, DH), hs3, degp,
                    b3.reshape(1, D), batch_p.reshape(N_PAD, 1))
    return out


# speedup vs baseline: 1.3357x; 1.0283x over previous
import functools

import jax
import jax.numpy as jnp
from jax import lax
from jax.experimental import pallas as pl
from jax.experimental.pallas import tpu as pltpu
from jax.experimental.pallas import tpu_sc as plsc

N = 10000
E = 320000
D = 128
G = 64
DH = D // 2

NC = 2
NS = 16
NW = NC * NS

CHUNK = 128
EPS_REAL = E // NS
NJE = 160
PADS = NJE * CHUNK - EPS_REAL
NJ = NJE // 2

N_PAD = 10240
TRASH = N_PAD - N
RPT = N_PAD // NS

NB = 5
LG = 3
ZR = 64

R = 2048

_mesh = plsc.VectorSubcoreMesh(core_axis_name="c", subcore_axis_name="s")


@functools.partial(
    pl.kernel,
    out_type=jax.ShapeDtypeStruct((NC * N_PAD,), jnp.float32),
    mesh=_mesh,
    scratch_types=[
        pltpu.VMEM((NJ, CHUNK), jnp.int32),
        pltpu.VMEM((CHUNK,), jnp.float32),
        pltpu.VMEM((RPT,), jnp.float32),
        pltpu.VMEM_SHARED((N_PAD,), jnp.float32),
    ],
)
def _deg_kernel(dst_hbm, out_hbm, idx_v, ones_v, zb_v, acc_sh):
    c = lax.axis_index("c")
    s = lax.axis_index("s")
    w = c * NS + s

    def zf(i, _):
        zb_v[pl.ds(i * 16, 16)] = jnp.zeros((16,), jnp.float32)
        return 0
    lax.fori_loop(0, RPT // 16, zf, 0)

    def of(i, _):
        ones_v[pl.ds(i * 16, 16)] = jnp.ones((16,), jnp.float32)
        return 0
    lax.fori_loop(0, CHUNK // 16, of, 0)

    pltpu.sync_copy(zb_v, acc_sh.at[pl.ds(s * RPT, RPT)])
    plsc.subcore_barrier()

    pltpu.sync_copy(dst_hbm.at[pl.ds(w * NJ, NJ)], idx_v)

    def body(j, _):
        pltpu.sync_copy(ones_v, acc_sh.at[idx_v.at[j]], add=True)
        return 0
    lax.fori_loop(0, NJ, body, 0)

    plsc.subcore_barrier()
    pltpu.sync_copy(acc_sh.at[pl.ds(s * RPT, RPT)],
                    out_hbm.at[pl.ds(c * N_PAD + s * RPT, RPT)])


@functools.partial(
    pl.kernel,
    out_type=jax.ShapeDtypeStruct((NC * N_PAD, DH), jnp.float32),
    mesh=_mesh,
    scratch_types=[
        pltpu.VMEM((NJE, CHUNK), jnp.int32),
        pltpu.VMEM((NJE, CHUNK), jnp.int32),
        [pltpu.VMEM((CHUNK, DH), jnp.float32)] * NB,
        pltpu.VMEM((ZR, DH), jnp.float32),
        [pltpu.SemaphoreType.DMA] * 2,
        [pltpu.SemaphoreType.DMA] * NB,
        [pltpu.SemaphoreType.DMA] * NB,
        pltpu.VMEM_SHARED((N_PAD, DH), jnp.float32),
    ],
    compiler_params=pltpu.CompilerParams(use_tc_tiling_on_sc=False),
)
def _agg_kernel(hs_hbm, src_hbm, dst_hbm, out_hbm, src_v, dst_v, rows, zb_v,
                isem, gsem, ssem, acc_sh):
    c = lax.axis_index("c")
    s = lax.axis_index("s")

    scpy = pltpu.make_async_copy(
        src_hbm.at[pl.ds((c * NS + s) * NJE, NJE)], src_v, isem[0])
    scpy.start()
    dcpy = pltpu.make_async_copy(
        dst_hbm.at[pl.ds(s * NJE, NJE)], dst_v, isem[1])
    dcpy.start()

    def gs(k, r):
        pltpu.async_copy(hs_hbm.at[src_v.at[k]], rows[r], gsem[r])

    def gw(k, r):
        pltpu.make_async_copy(hs_hbm.at[src_v.at[k]], rows[r], gsem[r]).wait()

    def ss(k, r):
        pltpu.async_copy(rows[r], acc_sh.at[dst_v.at[k]], ssem[r], add=True)

    def sw(k, r):
        pltpu.make_async_copy(rows[r], acc_sh.at[dst_v.at[k]], ssem[r]).wait()

    def zf(i, _):
        r = i // (DH // 16)
        col = (i % (DH // 16)) * 16
        zb_v[r, pl.ds(col, 16)] = jnp.zeros((16,), jnp.float32)
        return 0
    lax.fori_loop(0, ZR * DH // 16, zf, 0)

    scpy.wait()
    dcpy.wait()
    for r0 in range(LG):
        gs(r0, r0)

    def zcopy(t, _):
        pltpu.sync_copy(zb_v, acc_sh.at[pl.ds(s * RPT + t * ZR, ZR)])
        return 0
    lax.fori_loop(0, RPT // ZR, zcopy, 0)
    plsc.subcore_barrier()

    def ring(i, _):
        for b in range(NB):
            k = NB * i + b
            rb = (b + LG) % NB

            @pl.when(k >= NB - LG)
            def _():
                sw(k - (NB - LG), rb)

            @pl.when(k + LG < NJE)
            def _():
                gs(k + LG, rb)

            gw(k, b)
            ss(k, b)
        return 0
    lax.fori_loop(0, NJE // NB, ring, 0)

    for k in range(NJE - (NB - LG), NJE):
        sw(k, k % NB)

    plsc.subcore_barrier()

    def wb(t, _):
        pltpu.sync_copy(acc_sh.at[pl.ds(s * RPT + t * CHUNK, CHUNK)],
                        out_hbm.at[pl.ds(c * N_PAD + s * RPT + t * CHUNK, CHUNK)])
        return 0
    lax.fori_loop(0, RPT // CHUNK, wb, 0)


def _tc1_body(x_ref, w_ref, degp_ref, o_ref):
    j = pl.program_id(0)
    deg = degp_ref[0, pl.ds(j * R, R)] + degp_ref[1, pl.ds(j * R, R)] + 1.0
    dis = lax.rsqrt(deg)[:, None]
    h = jnp.dot(x_ref[...], w_ref[...], preferred_element_type=jnp.float32)
    h = h * dis
    o_ref[0] = h[:, :DH]
    o_ref[1] = h[:, DH:]


def _tc_mid_body(aggp_ref, hs_ref, degp_ref, w_ref, b_ref, o_ref):
    j = pl.program_id(0)
    deg = degp_ref[0, pl.ds(j * R, R)] + degp_ref[1, pl.ds(j * R, R)] + 1.0
    dis = lax.rsqrt(deg)[:, None]
    agg = jnp.concatenate([aggp_ref[0], aggp_ref[1]], axis=1)
    hs = jnp.concatenate([hs_ref[0], hs_ref[1]], axis=1)
    z = (agg + hs) * dis + b_ref[...]
    a = jnp.maximum(z, 0.0)
    h = jnp.dot(a, w_ref[...], preferred_element_type=jnp.float32) * dis
    o_ref[0] = h[:, :DH]
    o_ref[1] = h[:, DH:]


def _tc_final_body(aggp_ref, hs_ref, degp_ref, b_ref, batch_ref, o_ref):
    j = pl.program_id(0)
    deg = degp_ref[0, pl.ds(j * R, R)] + degp_ref[1, pl.ds(j * R, R)] + 1.0
    dis = lax.rsqrt(deg)[:, None]
    agg = jnp.concatenate([aggp_ref[0], aggp_ref[1]], axis=1)
    hs = jnp.concatenate([hs_ref[0], hs_ref[1]], axis=1)
    z = (agg + hs) * dis + b_ref[...]
    bb = batch_ref[...]
    oh = (bb == lax.broadcasted_iota(jnp.int32, (1, G), 1)).astype(jnp.float32)
    contrib = lax.dot_general(oh, z, (((0,), (0,)), ((), ())),
                              preferred_element_type=jnp.float32)

    @pl.when(j == 0)
    def _():
        o_ref[...] = jnp.zeros_like(o_ref)

    o_ref[...] += contrib


_x_spec = pl.BlockSpec((R, D), lambda j: (j, 0))
_w_spec = pl.BlockSpec((D, D), lambda j: (0, 0))
_degp_spec = pl.BlockSpec((2, N_PAD), lambda j: (0, 0))
_split_spec = pl.BlockSpec((2, R, DH), lambda j: (0, j, 0))
_b_spec = pl.BlockSpec((1, D), lambda j: (0, 0))

_split_shape = jax.ShapeDtypeStruct((2, N_PAD, DH), jnp.float32)

_tc1 = pl.pallas_call(
    _tc1_body,
    grid=(N_PAD // R,),
    in_specs=[_x_spec, _w_spec, _degp_spec],
    out_specs=_split_spec,
    out_shape=_split_shape,
)

_tc_mid = pl.pallas_call(
    _tc_mid_body,
    grid=(N_PAD // R,),
    in_specs=[_split_spec, _split_spec, _degp_spec, _w_spec, _b_spec],
    out_specs=_split_spec,
    out_shape=_split_shape,
)

_tc_final = pl.pallas_call(
    _tc_final_body,
    grid=(N_PAD // R,),
    in_specs=[_split_spec, _split_spec, _degp_spec, _b_spec,
              pl.BlockSpec((R, 1), lambda j: (j, 0))],
    out_specs=pl.BlockSpec((G, D), lambda j: (0, 0)),
    out_shape=jax.ShapeDtypeStruct((G, D), jnp.float32),
)


def kernel(x, edge_index, batch, W1, b1, W2, b2, W3, b3):
    src = edge_index[0].reshape(NS, EPS_REAL)
    dst = edge_index[1].reshape(NS, EPS_REAL)
    iw = jnp.arange(NS, dtype=jnp.int32)[:, None]
    ip = jnp.arange(PADS, dtype=jnp.int32)[None, :]
    pad_src = (iw * 613 + ip * 37) % N
    pad_dst = N + (iw * 7 + ip) % TRASH
    src_p = jnp.concatenate([src, pad_src], axis=1).reshape(NS * NJE, CHUNK)
    dst_p = jnp.concatenate([dst, pad_dst], axis=1).reshape(NS * NJE, CHUNK)
    src_p2 = jnp.concatenate([src_p, src_p + N_PAD], axis=0)

    degp = _deg_kernel(dst_p).reshape(NC, N_PAD)
    x_p = jnp.pad(x, ((0, N_PAD - N), (0, 0)))
    batch_p = jnp.pad(batch, (0, N_PAD - N), constant_values=G)
    hs1 = _tc1(x_p, W1, degp)
    agg1 = _agg_kernel(hs1.reshape(NC * N_PAD, DH), src_p2, dst_p)
    hs2 = _tc_mid(agg1.reshape(2, N_PAD, DH), hs1, degp, W2, b1.reshape(1, D))
    agg2 = _agg_kernel(hs2.reshape(NC * N_PAD, DH), src_p2, dst_p)
    hs3 = _tc_mid(agg2.reshape(2, N_PAD, DH), hs2, degp, W3, b2.reshape(1, D))
    agg3 = _agg_kernel(hs3.reshape(NC * N_PAD, DH), src_p2, dst_p)
    out = _tc_final(agg3.reshape(2, N_PAD, DH), hs3, degp,
                    b3.reshape(1, D), batch_p.reshape(N_PAD, 1))
    return out
